# Initial kernel scaffold; baseline (speedup 1.0000x reference)
import jax
import jax.numpy as jnp
from jax.experimental import pallas as pl


def kernel(lookup_ids, embedding_matrix):
    raise NotImplementedError("write your pallas kernel here")


# speedup vs baseline: 5.6560x; 5.6560x over previous
import functools

import jax
import jax.numpy as jnp
from jax import lax
from jax.experimental import pallas as pl
from jax.experimental.pallas import tpu as pltpu
from jax.experimental.pallas import tpu_sc as plsc

B, L, D, V = 512, 512, 128, 1024
PAD_ID, EOS_ID = 0, 1
N = B * L
NC, NS = 2, 16
NW = NC * NS
CHUNK = 128
CPW = N // (NW * CHUNK)

_mesh = plsc.VectorSubcoreMesh(core_axis_name="c", subcore_axis_name="s")


@functools.partial(
    pl.kernel,
    out_type=jax.ShapeDtypeStruct((N, D), jnp.float32),
    mesh=_mesh,
    scratch_types=[
        pltpu.VMEM((CPW, CHUNK), jnp.int32),
        pltpu.VMEM((2, CHUNK, D), jnp.float32),
        pltpu.SemaphoreType.DMA,
        pltpu.SemaphoreType.DMA,
        pltpu.SemaphoreType.DMA,
        pltpu.SemaphoreType.DMA,
    ],
)
def _gather_sc(ids_hbm, table_hbm, out_hbm, idx_v, rows_v, g0, g1, o0, o1):
    wid = lax.axis_index("s") * NC + lax.axis_index("c")
    base = wid * CPW
    pltpu.sync_copy(ids_hbm.at[pl.ds(base, CPW)], idx_v)

    def gather(j, slot, sem):
        return pltpu.async_copy(table_hbm.at[idx_v.at[j]], rows_v.at[slot], sem)

    def put(j, slot, sem):
        return pltpu.async_copy(
            rows_v.at[slot], out_hbm.at[pl.ds((base + j) * CHUNK, CHUNK)], sem)

    gather(0, 0, g0)
    gather(1, 1, g1)

    def body(i, _):
        j0 = 2 * i
        pltpu.make_async_copy(table_hbm.at[idx_v.at[j0]], rows_v.at[0], g0).wait()
        put(j0, 0, o0)
        pltpu.make_async_copy(table_hbm.at[idx_v.at[j0 + 1]], rows_v.at[1], g1).wait()
        put(j0 + 1, 1, o1)

        @pl.when(i + 1 < CPW // 2)
        def _():
            pltpu.make_async_copy(
                rows_v.at[0], out_hbm.at[pl.ds(0, CHUNK)], o0).wait()
            gather(j0 + 2, 0, g0)
            pltpu.make_async_copy(
                rows_v.at[1], out_hbm.at[pl.ds(0, CHUNK)], o1).wait()
            gather(j0 + 3, 1, g1)

        return 0

    lax.fori_loop(0, CPW // 2, body, 0)
    pltpu.make_async_copy(rows_v.at[0], out_hbm.at[pl.ds(0, CHUNK)], o0).wait()
    pltpu.make_async_copy(rows_v.at[1], out_hbm.at[pl.ds(0, CHUNK)], o1).wait()


def _mask_eos_body(ids_ref, mask_ref, eos_ref):
    ids = ids_ref[...]
    mask_ref[...] = ids == PAD_ID
    eos_ref[...] = (ids == EOS_ID).astype(jnp.float32)


_mask_eos = pl.pallas_call(
    _mask_eos_body,
    out_shape=(
        jax.ShapeDtypeStruct((B, L), jnp.bool_),
        jax.ShapeDtypeStruct((B, L), jnp.float32),
    ),
)


def kernel(lookup_ids, embedding_matrix):
    ids2d = lookup_ids.reshape(N // CHUNK, CHUNK)
    matrices = _gather_sc(ids2d, embedding_matrix).reshape(B, L, D)
    mask, eos = _mask_eos(lookup_ids)
    return (matrices, mask, eos, embedding_matrix, lookup_ids)


# speedup vs baseline: 8.9639x; 1.5848x over previous
import functools

import jax
import jax.numpy as jnp
from jax import lax
from jax.experimental import pallas as pl
from jax.experimental.pallas import tpu as pltpu
from jax.experimental.pallas import tpu_sc as plsc

B, L, D, V = 512, 512, 128, 1024
PAD_ID, EOS_ID = 0, 1
N = B * L
NC, NS = 2, 16
NW = NC * NS
CHUNK = 128
CPW = N // (NW * CHUNK)

_mesh = plsc.VectorSubcoreMesh(core_axis_name="c", subcore_axis_name="s")


@functools.partial(
    pl.kernel,
    out_type=jax.ShapeDtypeStruct((N, D), jnp.float32),
    mesh=_mesh,
    scratch_types=[
        pltpu.VMEM((CPW, CHUNK), jnp.int32),
        pltpu.VMEM((2, CHUNK, D), jnp.float32),
        pltpu.VMEM_SHARED((V, D), jnp.float32),
        pltpu.SemaphoreType.DMA,
        pltpu.SemaphoreType.DMA,
        pltpu.SemaphoreType.DMA,
        pltpu.SemaphoreType.DMA,
    ],
)
def _gather_sc(ids_hbm, table_hbm, out_hbm, idx_v, rows_v, tab_sh, g0, g1, o0, o1):
    wid = lax.axis_index("s") * NC + lax.axis_index("c")
    base = wid * CPW

    @pl.when(lax.axis_index("s") == 0)
    def _():
        pltpu.sync_copy(table_hbm, tab_sh)

    pltpu.sync_copy(ids_hbm.at[pl.ds(base, CPW)], idx_v)
    plsc.subcore_barrier()

    def gather(j, slot, sem):
        return pltpu.async_copy(tab_sh.at[idx_v.at[j]], rows_v.at[slot], sem)

    def put(j, slot, sem):
        return pltpu.async_copy(
            rows_v.at[slot], out_hbm.at[pl.ds((base + j) * CHUNK, CHUNK)], sem)

    gather(0, 0, g0)
    gather(1, 1, g1)

    def body(i, _):
        j0 = 2 * i
        pltpu.make_async_copy(tab_sh.at[idx_v.at[j0]], rows_v.at[0], g0).wait()
        put(j0, 0, o0)
        pltpu.make_async_copy(tab_sh.at[idx_v.at[j0 + 1]], rows_v.at[1], g1).wait()
        put(j0 + 1, 1, o1)

        @pl.when(i + 1 < CPW // 2)
        def _():
            pltpu.make_async_copy(
                rows_v.at[0], out_hbm.at[pl.ds(0, CHUNK)], o0).wait()
            gather(j0 + 2, 0, g0)
            pltpu.make_async_copy(
                rows_v.at[1], out_hbm.at[pl.ds(0, CHUNK)], o1).wait()
            gather(j0 + 3, 1, g1)

        return 0

    lax.fori_loop(0, CPW // 2, body, 0)
    pltpu.make_async_copy(rows_v.at[0], out_hbm.at[pl.ds(0, CHUNK)], o0).wait()
    pltpu.make_async_copy(rows_v.at[1], out_hbm.at[pl.ds(0, CHUNK)], o1).wait()


def _mask_eos_body(ids_ref, mask_ref, eos_ref):
    ids = ids_ref[...]
    mask_ref[...] = ids == PAD_ID
    eos_ref[...] = (ids == EOS_ID).astype(jnp.float32)


_mask_eos = pl.pallas_call(
    _mask_eos_body,
    out_shape=(
        jax.ShapeDtypeStruct((B, L), jnp.bool_),
        jax.ShapeDtypeStruct((B, L), jnp.float32),
    ),
)


def kernel(lookup_ids, embedding_matrix):
    ids2d = lookup_ids.reshape(N // CHUNK, CHUNK)
    matrices = _gather_sc(ids2d, embedding_matrix).reshape(B, L, D)
    mask, eos = _mask_eos(lookup_ids)
    return (matrices, mask, eos, embedding_matrix, lookup_ids)
